# Initial kernel scaffold; baseline (speedup 1.0000x reference)
import jax
import jax.numpy as jnp
from jax.experimental import pallas as pl


def kernel(x, edge_index, W_rel0, b_rel0, W_root0, W_rel1, b_rel1, W_root1):
    raise NotImplementedError("write your pallas kernel here")


# speedup vs baseline: 4.6169x; 4.6169x over previous
import functools

import jax
import jax.numpy as jnp
from jax import lax
from jax.experimental import pallas as pl
from jax.experimental.pallas import tpu as pltpu
from jax.experimental.pallas import tpu_sc as plsc

N = 10000
E = 320000
D = 128

NC = 2
NS = 16
LANES = 16

EPW = E // (NC * NS)
CHUNK = 80
NCHUNK = EPW // CHUNK
NPAD = 10240
ROWS_PW = NPAD // NS
ZROWS = 32


def _segsum_body(vals, srch, dsth, out, src_v, dst_v, rows_v, zb, agg, sem):
    c = lax.axis_index("c")
    s = lax.axis_index("s")

    zvec = jnp.zeros((LANES,), jnp.float32)

    def _zb_loop(t, _):
        i = t // (D // LANES)
        j = t % (D // LANES)
        zb[i, pl.ds(j * LANES, LANES)] = zvec
        return 0

    lax.fori_loop(0, ZROWS * (D // LANES), _zb_loop, 0)

    def _zero_loop(j, _):
        pltpu.sync_copy(zb, agg.at[pl.ds(s * ROWS_PW + j * ZROWS, ZROWS)])
        return 0

    lax.fori_loop(0, ROWS_PW // ZROWS, _zero_loop, 0)

    plsc.subcore_barrier()

    base = (c * NS + s) * EPW

    def _edge_loop(i, _):
        off = base + i * CHUNK
        pltpu.sync_copy(srch.at[pl.ds(off, CHUNK)], src_v)
        pltpu.sync_copy(dsth.at[pl.ds(off, CHUNK)], dst_v)
        pltpu.async_copy(vals.at[src_v], rows_v, sem).wait()
        pltpu.sync_copy(rows_v, agg.at[dst_v], add=True)
        return 0

    lax.fori_loop(0, NCHUNK, _edge_loop, 0)

    plsc.subcore_barrier()

    pltpu.sync_copy(
        agg.at[pl.ds(s * ROWS_PW, ROWS_PW)],
        out.at[c].at[pl.ds(s * ROWS_PW, ROWS_PW)],
    )


def _segsum_sc(vals, src, dst):
    mesh = plsc.VectorSubcoreMesh(
        core_axis_name="c", subcore_axis_name="s",
        num_cores=NC, num_subcores=NS,
    )
    f = pl.kernel(
        _segsum_body,
        out_type=jax.ShapeDtypeStruct((NC, NPAD, D), jnp.float32),
        mesh=mesh,
        scratch_types=[
            pltpu.VMEM((CHUNK,), jnp.int32),
            pltpu.VMEM((CHUNK,), jnp.int32),
            pltpu.VMEM((CHUNK, D), jnp.float32),
            pltpu.VMEM((ZROWS, D), jnp.float32),
            pltpu.VMEM_SHARED((NPAD, D), jnp.float32),
            pltpu.SemaphoreType.DMA,
        ],
    )
    return f(vals, src, dst)


def _dense_body(relu, a0, a1, xr, wrel, wroot, br, o):
    acc = jnp.dot(a0[...] + a1[...], wrel[...],
                  preferred_element_type=jnp.float32)
    acc += jnp.dot(xr[...], wroot[...], preferred_element_type=jnp.float32)
    acc += br[...]
    o[...] = jnp.maximum(acc, 0.0) if relu else acc


def _dense_tc(aggp, xin, wrel, b, wroot, relu):
    bm = 1000
    grid = (N // bm,)
    row_spec = pl.BlockSpec((bm, D), lambda i: (i, 0))
    w_spec = pl.BlockSpec((D, D), lambda i: (0, 0))
    return pl.pallas_call(
        functools.partial(_dense_body, relu),
        grid=grid,
        in_specs=[row_spec, row_spec, row_spec, w_spec, w_spec,
                  pl.BlockSpec((1, D), lambda i: (0, 0))],
        out_specs=row_spec,
        out_shape=jax.ShapeDtypeStruct((N, D), jnp.float32),
    )(aggp[0], aggp[1], xin, wrel, wroot, b.reshape(1, D))


def kernel(x, edge_index, W_rel0, b_rel0, W_root0, W_rel1, b_rel1, W_root1):
    src = edge_index[0]
    dst = edge_index[1]
    agg0 = _segsum_sc(x, src, dst)
    h = _dense_tc(agg0, x, W_rel0, b_rel0, W_root0, relu=True)
    agg1 = _segsum_sc(h, src, dst)
    out = _dense_tc(agg1, h, W_rel1, b_rel1, W_root1, relu=False)
    return out


# speedup vs baseline: 9.6695x; 2.0944x over previous
import functools

import jax
import jax.numpy as jnp
from jax import lax
from jax.experimental import pallas as pl
from jax.experimental.pallas import tpu as pltpu
from jax.experimental.pallas import tpu_sc as plsc

N = 10000
E = 320000
D = 128

NC = 2
NS = 16
LANES = 16

EPW = E // (NC * NS)
CHUNK = 128
NCHUNK = EPW // CHUNK
TAIL = EPW - NCHUNK * CHUNK
NPAD = 10240
ROWS_PW = NPAD // NS
ZROWS = 16


def _segsum_body(vals, srch, dsth, out,
                 dst_all, sv0, sv1, dv0, dv1, dvt, rows0, rows1, zb, agg,
                 dsem, isem0, isem1, gsem0, gsem1, ssem0, ssem1):
    c = lax.axis_index("c")
    s = lax.axis_index("s")
    rows = (rows0, rows1)
    sv = (sv0, sv1)
    dv = (dv0, dv1)
    isem = (isem0, isem1)
    gsem = (gsem0, gsem1)
    ssem = (ssem0, ssem1)

    base = (c * NS + s) * EPW

    cp_dst = pltpu.async_copy(dsth.at[pl.ds(base, EPW)], dst_all, dsem)
    pltpu.async_copy(srch.at[pl.ds(base, CHUNK)], sv0, isem0)
    pltpu.async_copy(srch.at[pl.ds(base + CHUNK, CHUNK)], sv1, isem1)

    zvec = jnp.zeros((LANES,), jnp.float32)

    def _zb_loop(t, _):
        i = t // (D // LANES)
        j = t % (D // LANES)
        zb[i, pl.ds(j * LANES, LANES)] = zvec
        return 0

    lax.fori_loop(0, ZROWS * (D // LANES), _zb_loop, 0)

    def _zero_loop(j, _):
        pltpu.sync_copy(zb, agg.at[pl.ds(s * ROWS_PW + j * ZROWS, ZROWS)])
        return 0

    lax.fori_loop(0, ROWS_PW // ZROWS, _zero_loop, 0)

    cp_dst.wait()
    plsc.subcore_barrier()

    def idx_copy(chunk_start, dvb, n):
        for j in range(n // LANES):
            dvb[pl.ds(j * LANES, LANES)] = (
                dst_all[pl.ds(chunk_start + j * LANES, LANES)])

    def issue_isrc(chunk, b):
        pltpu.async_copy(srch.at[pl.ds(base + chunk * CHUNK, CHUNK)],
                         sv[b], isem[b])

    def wait_isrc(b):
        pltpu.make_async_copy(srch.at[pl.ds(0, CHUNK)], sv[b],
                              isem[b]).wait()

    def issue_gather(b):
        pltpu.async_copy(vals.at[sv[b]], rows[b], gsem[b])

    def issue_scatter(b):
        pltpu.async_copy(rows[b], agg.at[dv[b]], ssem[b], add=True)

    def wait_g(b):
        pltpu.make_async_copy(vals.at[pl.ds(0, CHUNK)], rows[b],
                              gsem[b]).wait()

    def wait_s(b):
        pltpu.make_async_copy(vals.at[pl.ds(0, CHUNK)], rows[b],
                              ssem[b]).wait()

    idx_copy(0, dv[0], CHUNK)
    wait_isrc(0)
    issue_gather(0)

    def _pipe_body(i2, _):
        for b in range(2):
            i = 2 * i2 + b
            nb = 1 - b
            wait_g(b)
            issue_scatter(b)

            @pl.when(i + 2 < NCHUNK)
            def _():
                issue_isrc(i + 2, b)

            @pl.when(i >= 1)
            def _():
                wait_s(nb)

            @pl.when(i + 1 < NCHUNK)
            def _():
                idx_copy((i + 1) * CHUNK, dv[nb], CHUNK)
                wait_isrc(nb)
                issue_gather(nb)
        return 0

    lax.fori_loop(0, NCHUNK // 2, _pipe_body, 0)
    wait_s(1)

    idx_copy(NCHUNK * CHUNK, dvt, TAIL)
    pltpu.async_copy(srch.at[pl.ds(base + NCHUNK * CHUNK, TAIL)],
                     sv0.at[pl.ds(0, TAIL)], isem0)
    pltpu.make_async_copy(srch.at[pl.ds(0, TAIL)], sv0.at[pl.ds(0, TAIL)],
                          isem0).wait()
    pltpu.async_copy(vals.at[sv0.at[pl.ds(0, TAIL)]],
                     rows0.at[pl.ds(0, TAIL)], gsem0).wait()
    pltpu.sync_copy(rows0.at[pl.ds(0, TAIL)], agg.at[dvt], add=True)

    plsc.subcore_barrier()

    pltpu.sync_copy(
        agg.at[pl.ds(s * ROWS_PW, ROWS_PW)],
        out.at[c].at[pl.ds(s * ROWS_PW, ROWS_PW)],
    )


def _segsum_sc(vals, src, dst):
    mesh = plsc.VectorSubcoreMesh(
        core_axis_name="c", subcore_axis_name="s",
        num_cores=NC, num_subcores=NS,
    )
    f = pl.kernel(
        _segsum_body,
        out_type=jax.ShapeDtypeStruct((NC, NPAD, D), jnp.float32),
        mesh=mesh,
        scratch_types=[
            pltpu.VMEM((EPW,), jnp.int32),
            pltpu.VMEM((CHUNK,), jnp.int32),
            pltpu.VMEM((CHUNK,), jnp.int32),
            pltpu.VMEM((CHUNK,), jnp.int32),
            pltpu.VMEM((CHUNK,), jnp.int32),
            pltpu.VMEM((TAIL,), jnp.int32),
            pltpu.VMEM((CHUNK, D), jnp.float32),
            pltpu.VMEM((CHUNK, D), jnp.float32),
            pltpu.VMEM((ZROWS, D), jnp.float32),
            pltpu.VMEM_SHARED((NPAD, D), jnp.float32),
            pltpu.SemaphoreType.DMA,
            pltpu.SemaphoreType.DMA,
            pltpu.SemaphoreType.DMA,
            pltpu.SemaphoreType.DMA,
            pltpu.SemaphoreType.DMA,
            pltpu.SemaphoreType.DMA,
            pltpu.SemaphoreType.DMA,
        ],
    )
    return f(vals, src, dst)


def _dense_body(relu, a0, a1, xr, wrel, wroot, br, o):
    acc = jnp.dot(a0[...] + a1[...], wrel[...],
                  preferred_element_type=jnp.float32)
    acc += jnp.dot(xr[...], wroot[...], preferred_element_type=jnp.float32)
    acc += br[...]
    o[...] = jnp.maximum(acc, 0.0) if relu else acc


def _dense_tc(aggp, xin, wrel, b, wroot, relu):
    bm = 1000
    grid = (N // bm,)
    row_spec = pl.BlockSpec((bm, D), lambda i: (i, 0))
    w_spec = pl.BlockSpec((D, D), lambda i: (0, 0))
    return pl.pallas_call(
        functools.partial(_dense_body, relu),
        grid=grid,
        in_specs=[row_spec, row_spec, row_spec, w_spec, w_spec,
                  pl.BlockSpec((1, D), lambda i: (0, 0))],
        out_specs=row_spec,
        out_shape=jax.ShapeDtypeStruct((N, D), jnp.float32),
    )(aggp[0], aggp[1], xin, wrel, wroot, b.reshape(1, D))


def kernel(x, edge_index, W_rel0, b_rel0, W_root0, W_rel1, b_rel1, W_root1):
    src = edge_index[0]
    dst = edge_index[1]
    agg0 = _segsum_sc(x, src, dst)
    h = _dense_tc(agg0, x, W_rel0, b_rel0, W_root0, relu=True)
    agg1 = _segsum_sc(h, src, dst)
    out = _dense_tc(agg1, h, W_rel1, b_rel1, W_root1, relu=False)
    return out
